# baseline (speedup 1.0000x reference)
import functools
import math

import jax
import jax.numpy as jnp
from jax import lax
from jax.experimental import pallas as pl
from jax.experimental.pallas import tpu as pltpu
from jax.experimental.pallas import tpu_sc as plsc

L = 16
NC, NS = 2, 16
NW = NC * NS
CHUNK = 32000


def _take16(vec, idx):
    return jnp.take_along_axis(vec, idx, axis=0, mode="promise_in_bounds")


def _search_and_eval(x, t_lo, t_l5, t_ref, a_ref, b_ref, c_ref, d_ref, depth):
    i = jnp.ones((L,), jnp.int32)
    for _ in range(4):
        v = _take16(t_lo, i)
        i = i + i + (v < x).astype(jnp.int32)
    v = _take16(t_l5, i - L)
    i = i + i + (v < x).astype(jnp.int32)
    for _ in range(depth - 5):
        v = plsc.load_gather(t_ref, [i])
        i = i + i + (v < x).astype(jnp.int32)
    idx = i - (1 << depth)
    av = plsc.load_gather(a_ref, [idx])
    bv = plsc.load_gather(b_ref, [idx])
    cv = plsc.load_gather(c_ref, [idx])
    dv = plsc.load_gather(d_ref, [idx])
    return ((dv * x + cv) * x + bv) * x + av


def _make_sc_kernel(n, k):
    assert n % CHUNK == 0 and CHUNK % L == 0
    n_chunks = n // CHUNK
    chunks_per_worker = -(-n_chunks // NW)
    n_vec = CHUNK // L
    depth = max(5, math.ceil(math.log2(k - 1)))
    tsize = 1 << depth

    mesh = plsc.VectorSubcoreMesh(core_axis_name="c", subcore_axis_name="s")

    @functools.partial(
        pl.kernel,
        out_type=jax.ShapeDtypeStruct((n,), jnp.float32),
        mesh=mesh,
        compiler_params=pltpu.CompilerParams(needs_layout_passes=False,
                                             use_tc_tiling_on_sc=False),
        scratch_types=[
            pltpu.VMEM((CHUNK,), jnp.float32),
            pltpu.VMEM((CHUNK,), jnp.float32),
            pltpu.VMEM((CHUNK,), jnp.float32),
            pltpu.VMEM((tsize,), jnp.float32),
            pltpu.VMEM((tsize,), jnp.float32),
        ] + [pltpu.VMEM((k,), jnp.float32) for _ in range(8)],
    )
    def sc_kernel(x0_hbm, x1_hbm, t0_hbm, t1_hbm, a0_hbm, b0_hbm, c0_hbm,
                  d0_hbm, a1_hbm, b1_hbm, c1_hbm, d1_hbm, out_hbm,
                  x0_v, x1_v, out_v, t0_v, t1_v, a0_v, b0_v, c0_v, d0_v,
                  a1_v, b1_v, c1_v, d1_v):
        wid = lax.axis_index("s") * NC + lax.axis_index("c")

        pltpu.sync_copy(t0_hbm, t0_v)
        pltpu.sync_copy(t1_hbm, t1_v)
        pltpu.sync_copy(a0_hbm, a0_v)
        pltpu.sync_copy(b0_hbm, b0_v)
        pltpu.sync_copy(c0_hbm, c0_v)
        pltpu.sync_copy(d0_hbm, d0_v)
        pltpu.sync_copy(a1_hbm, a1_v)
        pltpu.sync_copy(b1_hbm, b1_v)
        pltpu.sync_copy(c1_hbm, c1_v)
        pltpu.sync_copy(d1_hbm, d1_v)

        t0_lo = t0_v[pl.ds(0, L)]
        t0_l5 = t0_v[pl.ds(L, L)]
        t1_lo = t1_v[pl.ds(0, L)]
        t1_l5 = t1_v[pl.ds(L, L)]

        def chunk_body(c, _):
            chunk_id = wid + c * NW

            @pl.when(chunk_id < n_chunks)
            def _():
                base = chunk_id * CHUNK
                pltpu.sync_copy(x0_hbm.at[pl.ds(base, CHUNK)], x0_v)
                pltpu.sync_copy(x1_hbm.at[pl.ds(base, CHUNK)], x1_v)

                @plsc.parallel_loop(0, n_vec, unroll=10)
                def vec_body0(v):
                    x0 = x0_v[pl.ds(v * L, L)]
                    p0 = _search_and_eval(x0, t0_lo, t0_l5, t0_v, a0_v,
                                          b0_v, c0_v, d0_v, depth)
                    out_v[pl.ds(v * L, L)] = p0

                @plsc.parallel_loop(0, n_vec, unroll=10)
                def vec_body1(v):
                    x1 = x1_v[pl.ds(v * L, L)]
                    p1 = _search_and_eval(x1, t1_lo, t1_l5, t1_v, a1_v,
                                          b1_v, c1_v, d1_v, depth)
                    out_v[pl.ds(v * L, L)] = out_v[pl.ds(v * L, L)] * p1
                pltpu.sync_copy(out_v, out_hbm.at[pl.ds(base, CHUNK)])

            return _

        lax.fori_loop(0, chunks_per_worker, chunk_body, None)

    return sc_kernel


def _eytzinger_perm(depth):
    size = 1 << depth
    perm = [0] * size
    stack = [(0, size - 2, 1)]
    while stack:
        lo, hi, i = stack.pop()
        if lo > hi:
            continue
        mid = (lo + hi) // 2
        perm[i] = mid
        stack.append((lo, mid - 1, 2 * i))
        stack.append((mid + 1, hi, 2 * i + 1))
    return perm


def kernel(x, knots, a, b, c, d):
    n = x.shape[0]
    k = knots.shape[0]
    depth = max(5, math.ceil(math.log2(k - 1)))
    tsize = 1 << depth
    pad = jnp.full((tsize - 1 - (k - 2),), jnp.inf, jnp.float32)
    perm = jnp.asarray(_eytzinger_perm(depth)[1:], jnp.int32)

    def tree(j):
        srt = jnp.concatenate([knots[1:k - 1, j], pad])
        return jnp.concatenate([jnp.zeros((1,), jnp.float32), srt[perm]])

    def col(t, j):
        return jnp.concatenate([t[:, j], jnp.zeros((1,), jnp.float32)])

    sc = _make_sc_kernel(n, k)
    return sc(x[:, 0], x[:, 1], tree(0), tree(1),
              col(a, 0), col(b, 0), col(c, 0), col(d, 0),
              col(a, 1), col(b, 1), col(c, 1), col(d, 1))


# speedup vs baseline: 1.0431x; 1.0431x over previous
import functools
import math

import jax
import jax.numpy as jnp
from jax import lax
from jax.experimental import pallas as pl
from jax.experimental.pallas import tpu as pltpu
from jax.experimental.pallas import tpu_sc as plsc

L = 16
NC, NS = 2, 16
NW = NC * NS
CHUNK = 32000


def _take16(vec, idx):
    return jnp.take_along_axis(vec, idx, axis=0, mode="promise_in_bounds")


def _search_and_eval(x, t_lo, t_l5, t_ref, a_ref, b_ref, c_ref, d_ref, depth):
    i = jnp.ones((L,), jnp.int32)
    for _ in range(4):
        v = _take16(t_lo, i)
        i = i + i + (v < x).astype(jnp.int32)
    v = _take16(t_l5, i - L)
    i = i + i + (v < x).astype(jnp.int32)
    for _ in range(depth - 5):
        v = plsc.load_gather(t_ref, [i])
        i = i + i + (v < x).astype(jnp.int32)
    idx = i - (1 << depth)
    av = plsc.load_gather(a_ref, [idx])
    bv = plsc.load_gather(b_ref, [idx])
    cv = plsc.load_gather(c_ref, [idx])
    dv = plsc.load_gather(d_ref, [idx])
    return ((dv * x + cv) * x + bv) * x + av


def _make_sc_kernel(n, k):
    assert n % CHUNK == 0 and CHUNK % L == 0
    n_chunks = n // CHUNK
    chunks_per_worker = -(-n_chunks // NW)
    n_vec = CHUNK // L
    depth = max(5, math.ceil(math.log2(k - 1)))
    tsize = 1 << depth

    mesh = plsc.VectorSubcoreMesh(core_axis_name="c", subcore_axis_name="s")

    @functools.partial(
        pl.kernel,
        out_type=jax.ShapeDtypeStruct((n,), jnp.float32),
        mesh=mesh,
        compiler_params=pltpu.CompilerParams(needs_layout_passes=False,
                                             use_tc_tiling_on_sc=False),
        scratch_types=[
            pltpu.VMEM((CHUNK,), jnp.float32),
            pltpu.VMEM((CHUNK,), jnp.float32),
            pltpu.VMEM((CHUNK,), jnp.float32),
            pltpu.VMEM((tsize,), jnp.float32),
            pltpu.VMEM((tsize,), jnp.float32),
        ] + [pltpu.VMEM((k,), jnp.float32) for _ in range(8)],
    )
    def sc_kernel(x0_hbm, x1_hbm, t0_hbm, t1_hbm, a0_hbm, b0_hbm, c0_hbm,
                  d0_hbm, a1_hbm, b1_hbm, c1_hbm, d1_hbm, out_hbm,
                  x0_v, x1_v, out_v, t0_v, t1_v, a0_v, b0_v, c0_v, d0_v,
                  a1_v, b1_v, c1_v, d1_v):
        wid = lax.axis_index("s") * NC + lax.axis_index("c")

        pltpu.sync_copy(t0_hbm, t0_v)
        pltpu.sync_copy(t1_hbm, t1_v)
        pltpu.sync_copy(a0_hbm, a0_v)
        pltpu.sync_copy(b0_hbm, b0_v)
        pltpu.sync_copy(c0_hbm, c0_v)
        pltpu.sync_copy(d0_hbm, d0_v)
        pltpu.sync_copy(a1_hbm, a1_v)
        pltpu.sync_copy(b1_hbm, b1_v)
        pltpu.sync_copy(c1_hbm, c1_v)
        pltpu.sync_copy(d1_hbm, d1_v)

        t0_lo = t0_v[pl.ds(0, L)]
        t0_l5 = t0_v[pl.ds(L, L)]
        t1_lo = t1_v[pl.ds(0, L)]
        t1_l5 = t1_v[pl.ds(L, L)]

        def chunk_body(c, _):
            chunk_id = wid + c * NW

            @pl.when(chunk_id < n_chunks)
            def _():
                base = chunk_id * CHUNK
                pltpu.sync_copy(x0_hbm.at[pl.ds(base, CHUNK)], x0_v)
                pltpu.sync_copy(x1_hbm.at[pl.ds(base, CHUNK)], x1_v)

                @plsc.parallel_loop(0, n_vec, unroll=10)
                def vec_body(v):
                    x0 = x0_v[pl.ds(v * L, L)]
                    x1 = x1_v[pl.ds(v * L, L)]
                    p0 = _search_and_eval(x0, t0_lo, t0_l5, t0_v, a0_v,
                                          b0_v, c0_v, d0_v, depth)
                    p1 = _search_and_eval(x1, t1_lo, t1_l5, t1_v, a1_v,
                                          b1_v, c1_v, d1_v, depth)
                    out_v[pl.ds(v * L, L)] = p0 * p1
                pltpu.sync_copy(out_v, out_hbm.at[pl.ds(base, CHUNK)])

            return _

        lax.fori_loop(0, chunks_per_worker, chunk_body, None)

    return sc_kernel


def _eytzinger_perm(depth):
    size = 1 << depth
    perm = [0] * size
    stack = [(0, size - 2, 1)]
    while stack:
        lo, hi, i = stack.pop()
        if lo > hi:
            continue
        mid = (lo + hi) // 2
        perm[i] = mid
        stack.append((lo, mid - 1, 2 * i))
        stack.append((mid + 1, hi, 2 * i + 1))
    return perm


def kernel(x, knots, a, b, c, d):
    n = x.shape[0]
    k = knots.shape[0]
    depth = max(5, math.ceil(math.log2(k - 1)))
    tsize = 1 << depth
    pad = jnp.full((tsize - 1 - (k - 2),), jnp.inf, jnp.float32)
    perm = jnp.asarray(_eytzinger_perm(depth)[1:], jnp.int32)

    def tree(j):
        srt = jnp.concatenate([knots[1:k - 1, j], pad])
        return jnp.concatenate([jnp.zeros((1,), jnp.float32), srt[perm]])

    def col(t, j):
        return jnp.concatenate([t[:, j], jnp.zeros((1,), jnp.float32)])

    sc = _make_sc_kernel(n, k)
    return sc(x[:, 0], x[:, 1], tree(0), tree(1),
              col(a, 0), col(b, 0), col(c, 0), col(d, 0),
              col(a, 1), col(b, 1), col(c, 1), col(d, 1))


# speedup vs baseline: 1.3288x; 1.2739x over previous
import functools
import math

import jax
import jax.numpy as jnp
from jax import lax
from jax.experimental import pallas as pl
from jax.experimental.pallas import tpu as pltpu
from jax.experimental.pallas import tpu_sc as plsc

L = 16
NC, NS = 2, 16
NW = NC * NS
CHUNK = 32000


def _take16(vec, idx):
    return jnp.take_along_axis(vec, idx, axis=0, mode="promise_in_bounds")


def _search_and_eval(x, t_lo, t_l5, t_ref, a_ref, b_ref, c_ref, d_ref, depth):
    i = jnp.ones((L,), jnp.int32)
    for _ in range(4):
        v = _take16(t_lo, i)
        i = i + i + (v < x).astype(jnp.int32)
    v = _take16(t_l5, i - L)
    i = i + i + (v < x).astype(jnp.int32)
    for _ in range(depth - 5):
        v = plsc.load_gather(t_ref, [i])
        i = i + i + (v < x).astype(jnp.int32)
    idx = i - (1 << depth)
    av = plsc.load_gather(a_ref, [idx])
    bv = plsc.load_gather(b_ref, [idx])
    cv = plsc.load_gather(c_ref, [idx])
    dv = plsc.load_gather(d_ref, [idx])
    return ((dv * x + cv) * x + bv) * x + av


def _make_sc_kernel(n, k):
    assert n % CHUNK == 0 and CHUNK % L == 0
    n_chunks = n // CHUNK
    chunks_per_worker = -(-n_chunks // NW)
    n_vec = CHUNK // L
    depth = max(5, math.ceil(math.log2(k - 1)))
    tsize = 1 << depth

    mesh = plsc.VectorSubcoreMesh(core_axis_name="c", subcore_axis_name="s")

    @functools.partial(
        pl.kernel,
        out_type=jax.ShapeDtypeStruct((n,), jnp.float32),
        mesh=mesh,
        compiler_params=pltpu.CompilerParams(needs_layout_passes=False,
                                             use_tc_tiling_on_sc=False),
        scratch_types=[
            pltpu.VMEM((CHUNK,), jnp.float32),
            pltpu.VMEM((CHUNK,), jnp.float32),
            pltpu.VMEM((CHUNK,), jnp.float32),
            pltpu.VMEM((tsize,), jnp.float32),
            pltpu.VMEM((tsize,), jnp.float32),
        ] + [pltpu.VMEM((k,), jnp.float32) for _ in range(8)],
    )
    def sc_kernel(x0_hbm, x1_hbm, t0_hbm, t1_hbm, a0_hbm, b0_hbm, c0_hbm,
                  d0_hbm, a1_hbm, b1_hbm, c1_hbm, d1_hbm, out_hbm,
                  x0_v, x1_v, out_v, t0_v, t1_v, a0_v, b0_v, c0_v, d0_v,
                  a1_v, b1_v, c1_v, d1_v):
        wid = lax.axis_index("s") * NC + lax.axis_index("c")

        pltpu.sync_copy(t0_hbm, t0_v)
        pltpu.sync_copy(t1_hbm, t1_v)
        pltpu.sync_copy(a0_hbm, a0_v)
        pltpu.sync_copy(b0_hbm, b0_v)
        pltpu.sync_copy(c0_hbm, c0_v)
        pltpu.sync_copy(d0_hbm, d0_v)
        pltpu.sync_copy(a1_hbm, a1_v)
        pltpu.sync_copy(b1_hbm, b1_v)
        pltpu.sync_copy(c1_hbm, c1_v)
        pltpu.sync_copy(d1_hbm, d1_v)

        t0_lo = t0_v[pl.ds(0, L)]
        t0_l5 = t0_v[pl.ds(L, L)]
        t1_lo = t1_v[pl.ds(0, L)]
        t1_l5 = t1_v[pl.ds(L, L)]

        def chunk_body(c, _):
            chunk_id = wid + c * NW

            @pl.when(chunk_id < n_chunks)
            def _():
                base = chunk_id * CHUNK
                pltpu.sync_copy(x0_hbm.at[pl.ds(base, CHUNK)], x0_v)
                pltpu.sync_copy(x1_hbm.at[pl.ds(base, CHUNK)], x1_v)

                @plsc.parallel_loop(0, n_vec, unroll=10)
                def vec_body(v):
                    x0 = x0_v[pl.ds(v * L, L)]
                    x1 = x1_v[pl.ds(v * L, L)]
                    p0 = _search_and_eval(x0, t0_lo, t0_l5, t0_v, a0_v,
                                          b0_v, c0_v, d0_v, depth)
                    p1 = _search_and_eval(x1, t1_lo, t1_l5, t1_v, a1_v,
                                          b1_v, c1_v, d1_v, depth)
                    out_v[pl.ds(v * L, L)] = p0 * p1
                pltpu.sync_copy(out_v, out_hbm.at[pl.ds(base, CHUNK)])

            return _

        lax.fori_loop(0, chunks_per_worker, chunk_body, None)

    return sc_kernel


def _deinterleave_tc(xt):
    n = xt.shape[1]
    blk = 32000
    assert n % blk == 0

    def body(x_ref, o0_ref, o1_ref):
        i = pl.program_id(0)
        o0_ref[pl.ds(i * blk, blk)] = x_ref[0, :]
        o1_ref[pl.ds(i * blk, blk)] = x_ref[1, :]

    return pl.pallas_call(
        body,
        grid=(n // blk,),
        in_specs=[pl.BlockSpec((2, blk), lambda i: (0, i))],
        out_specs=[pl.BlockSpec((n,), lambda i: (0,)),
                   pl.BlockSpec((n,), lambda i: (0,))],
        out_shape=[jax.ShapeDtypeStruct((n,), jnp.float32),
                   jax.ShapeDtypeStruct((n,), jnp.float32)],
    )(xt)


def _eytzinger_perm(depth):
    size = 1 << depth
    perm = [0] * size
    stack = [(0, size - 2, 1)]
    while stack:
        lo, hi, i = stack.pop()
        if lo > hi:
            continue
        mid = (lo + hi) // 2
        perm[i] = mid
        stack.append((lo, mid - 1, 2 * i))
        stack.append((mid + 1, hi, 2 * i + 1))
    return perm


def kernel(x, knots, a, b, c, d):
    n = x.shape[0]
    k = knots.shape[0]
    depth = max(5, math.ceil(math.log2(k - 1)))
    tsize = 1 << depth
    pad = jnp.full((tsize - 1 - (k - 2),), jnp.inf, jnp.float32)
    perm = jnp.asarray(_eytzinger_perm(depth)[1:], jnp.int32)

    def tree(j):
        srt = jnp.concatenate([knots[1:k - 1, j], pad])
        return jnp.concatenate([jnp.zeros((1,), jnp.float32), srt[perm]])

    def col(t, j):
        return jnp.concatenate([t[:, j], jnp.zeros((1,), jnp.float32)])

    x0, x1 = _deinterleave_tc(x.T)
    sc = _make_sc_kernel(n, k)
    return sc(x0, x1, tree(0), tree(1),
              col(a, 0), col(b, 0), col(c, 0), col(d, 0),
              col(a, 1), col(b, 1), col(c, 1), col(d, 1))
